# baseline (speedup 1.0000x reference)
import functools
import math

import jax
import jax.numpy as jnp
import numpy as np
from jax import lax
from jax.experimental import pallas as pl
from jax.experimental.pallas import tpu as pltpu
from jax.experimental.pallas import tpu_sc as plsc

_VOCAB_TITLES = 1000000
_GENRE_VOCAB = 51
_EMBED = 32
_BATCH = 16384
_N_GENRES = 5
_ADAPT = np.array([1.0, 1.5, 2.0, 2.5, 3.0, 3.5, 4.0, 4.5, 5.0], dtype=np.float32)
_NORM_MEAN = float(_ADAPT.mean())
_INV_STD = float(1.0 / math.sqrt(float(_ADAPT.var())))

_OUT_W = 2 * _EMBED + 1

_info = plsc.get_sparse_core_info()
_NC, _NS, _L = _info.num_cores, _info.num_subcores, _info.num_lanes
_NW = _NC * _NS
_BW = _BATCH // _NW

_V = _VOCAB_TITLES + 1
_VPAD = ((_V + 7) // 8) * 8
_CHUNK = 32768
_NFULL = 30
_CHUNK2 = 16896
_NTAIL = _V - _NFULL * _CHUNK - _CHUNK2
_TPAD = 72
_NCHUNKS = _NFULL + 1
_BPIECE = _BW // _NCHUNKS + 1


def _fmt_body(tabT_hbm, tail_hbm, title_hbm, gidx_hbm, rating_hbm, gtab_hbm,
              flat_hbm, fidx_hbm, part_hbm,
              buf0, buf1, tail_v, idx_v, fidx_v, gtab_v, gidx_v, rate_v,
              part_v, sem0, sem1):
    d = lax.axis_index("s") * _NC + lax.axis_index("c")
    base = d * _BW
    bufs = (buf0, buf1)
    sems = (sem0, sem1)

    prev = pltpu.async_copy(tabT_hbm.at[d, pl.ds(0, _CHUNK)],
                            buf0.at[pl.ds(0, _CHUNK)], sem0)

    pltpu.sync_copy(tail_hbm, tail_v)
    pltpu.sync_copy(title_hbm.at[pl.ds(base, _BW)], idx_v.at[pl.ds(0, _BW)])
    pltpu.sync_copy(gtab_hbm, gtab_v)
    pltpu.sync_copy(gidx_hbm.at[pl.ds(base * _N_GENRES, _BW * _N_GENRES)],
                    gidx_v.at[pl.ds(0, _BW * _N_GENRES)])
    pltpu.sync_copy(rating_hbm.at[pl.ds(base, _BW)], rate_v.at[pl.ds(0, _BW)])

    pltpu.sync_copy(tail_v.at[pl.ds(d * _TPAD, _NTAIL)],
                    flat_hbm.at[pl.ds(d * _VPAD + _NFULL * _CHUNK + _CHUNK2,
                                      _NTAIL)])

    lanes = lax.iota(jnp.int32, _L)
    dlo = lanes * _VPAD
    dhi = (lanes + _L) * _VPAD

    def batch_body(b, carry):
        r16 = idx_v[pl.ds(b, _L)]
        r = r16[0]
        fidx_v[pl.ds(b * _EMBED, _L)] = dlo + r
        fidx_v[pl.ds(b * _EMBED + _L, _L)] = dhi + r
        gids = gidx_v[pl.ds(b * _N_GENRES, _L)]
        g0 = jnp.zeros((_L,), jnp.float32)
        g1 = jnp.zeros((_L,), jnp.float32)
        for k in range(_N_GENRES):
            gid = gids[k]
            g0 = g0 + gtab_v[gid, pl.ds(0, _L)]
            g1 = g1 + gtab_v[gid, pl.ds(_L, _L)]
        part_v[pl.ds(b * _OUT_W + _EMBED, _L)] = g0 * (1.0 / _N_GENRES)
        r0 = rate_v[pl.ds(b, _L)][0]
        rn = (r0 - _NORM_MEAN) * _INV_STD
        part_v[pl.ds(b * _OUT_W + _EMBED + _L, _L)] = g1 * (1.0 / _N_GENRES)
        w = part_v[pl.ds(b * _OUT_W + _OUT_W - _L, _L)]
        part_v[pl.ds(b * _OUT_W + _OUT_W - _L, _L)] = jnp.where(
            lanes == _L - 1, rn, w)
        return carry

    sizes = [_CHUNK] * _NFULL + [_CHUNK2]
    offs = [k * _CHUNK for k in range(_NFULL)] + [_NFULL * _CHUNK]
    done = 0
    for k in range(_NCHUNKS):
        nxt = None
        if k + 1 < _NCHUNKS:
            nxt = pltpu.async_copy(
                tabT_hbm.at[d, pl.ds(offs[k + 1], sizes[k + 1])],
                bufs[(k + 1) % 2].at[pl.ds(0, sizes[k + 1])],
                sems[(k + 1) % 2])
        hi = min(done + _BPIECE, _BW)
        if done < hi:
            lax.fori_loop(done, hi, batch_body, 0)
            done = hi
        prev.wait()
        pltpu.sync_copy(bufs[k % 2].at[pl.ds(0, sizes[k])],
                        flat_hbm.at[pl.ds(d * _VPAD + offs[k], sizes[k])])
        prev = nxt
    if done < _BW:
        lax.fori_loop(done, _BW, batch_body, 0)

    pltpu.sync_copy(fidx_v, fidx_hbm.at[pl.ds(base * _EMBED, _BW * _EMBED)])
    pltpu.sync_copy(part_v, part_hbm.at[pl.ds(base * _OUT_W, _BW * _OUT_W)])


def _gather_body(ttab_hbm, fidx_hbm, part_hbm, out_hbm,
                 fidx_v, trows_v, out_v, sem):
    wid = lax.axis_index("s") * _NC + lax.axis_index("c")
    base = wid * _BW

    pltpu.sync_copy(fidx_hbm.at[pl.ds(base * _EMBED, _BW * _EMBED)], fidx_v)
    title_dma = pltpu.async_copy(ttab_hbm.at[fidx_v], trows_v, sem)
    pltpu.sync_copy(part_hbm.at[pl.ds(base * _OUT_W, _BW * _OUT_W)], out_v)
    title_dma.wait()

    def title_body(b, carry):
        out_v[pl.ds(b * _OUT_W, _L)] = trows_v[pl.ds(b * _EMBED, _L)]
        out_v[pl.ds(b * _OUT_W + _L, _L)] = trows_v[pl.ds(b * _EMBED + _L, _L)]
        return carry

    lax.fori_loop(0, _BW, title_body, 0)

    pltpu.sync_copy(out_v, out_hbm.at[pl.ds(base * _OUT_W, _BW * _OUT_W)])


def kernel(title, book_genres, bucketized_average_rating, title_table, genre_table):
    gidx_flat = book_genres.reshape(-1)
    tabT = title_table.T
    tail72 = jnp.pad(tabT[:, _V - _NTAIL:], ((0, 0), (0, _TPAD - _NTAIL)))
    mesh = plsc.VectorSubcoreMesh(core_axis_name="c", subcore_axis_name="s")

    fmt = functools.partial(
        pl.kernel,
        mesh=mesh,
        out_type=(
            jax.ShapeDtypeStruct((_EMBED * _VPAD,), jnp.float32),
            jax.ShapeDtypeStruct((_BATCH * _EMBED,), jnp.int32),
            jax.ShapeDtypeStruct((_BATCH * _OUT_W,), jnp.float32),
        ),
        scratch_types=[
            pltpu.VMEM((_CHUNK,), jnp.float32),
            pltpu.VMEM((_CHUNK,), jnp.float32),
            pltpu.VMEM((_EMBED * _TPAD,), jnp.float32),
            pltpu.VMEM((_BW + _L,), jnp.int32),
            pltpu.VMEM((_BW * _EMBED,), jnp.int32),
            pltpu.VMEM((_GENRE_VOCAB, _EMBED), jnp.float32),
            pltpu.VMEM((_BW * _N_GENRES + _L,), jnp.int32),
            pltpu.VMEM((_BW + _L,), jnp.float32),
            pltpu.VMEM((_BW * _OUT_W,), jnp.float32),
            pltpu.SemaphoreType.DMA,
            pltpu.SemaphoreType.DMA,
        ],
    )(_fmt_body)
    ttab_flat, fidx, part = fmt(tabT, tail72.reshape(-1), title, gidx_flat,
                                bucketized_average_rating, genre_table)

    run = functools.partial(
        pl.kernel,
        mesh=mesh,
        compiler_params=pltpu.CompilerParams(use_tc_tiling_on_sc=False),
        out_type=jax.ShapeDtypeStruct((_BATCH * _OUT_W,), jnp.float32),
        scratch_types=[
            pltpu.VMEM((_BW * _EMBED,), jnp.int32),
            pltpu.VMEM((_BW * _EMBED,), jnp.float32),
            pltpu.VMEM((_BW * _OUT_W,), jnp.float32),
            pltpu.SemaphoreType.DMA,
        ],
    )(_gather_body)
    out_flat = run(ttab_flat, fidx, part)
    return out_flat.reshape(_BATCH, _OUT_W)


# speedup vs baseline: 1.0462x; 1.0462x over previous
import functools
import math

import jax
import jax.numpy as jnp
import numpy as np
from jax import lax
from jax.experimental import pallas as pl
from jax.experimental.pallas import tpu as pltpu
from jax.experimental.pallas import tpu_sc as plsc

_VOCAB_TITLES = 1000000
_GENRE_VOCAB = 51
_EMBED = 32
_BATCH = 16384
_N_GENRES = 5
_ADAPT = np.array([1.0, 1.5, 2.0, 2.5, 3.0, 3.5, 4.0, 4.5, 5.0], dtype=np.float32)
_NORM_MEAN = float(_ADAPT.mean())
_INV_STD = float(1.0 / math.sqrt(float(_ADAPT.var())))

_OUT_W = 2 * _EMBED + 1

_info = plsc.get_sparse_core_info()
_NC, _NS, _L = _info.num_cores, _info.num_subcores, _info.num_lanes
_NW = _NC * _NS
_BW = _BATCH // _NW

_V = _VOCAB_TITLES + 1
_VPAD = ((_V + 7) // 8) * 8
_CHUNK = 32768
_NFULL = 30
_CHUNK2 = 16896
_NTAIL = _V - _NFULL * _CHUNK - _CHUNK2
_TPAD = 72


def _fmt_body(tabT_hbm, tail_hbm, flat_hbm, buf0, buf1, buf2, tail_v,
              rs0, rs1, rs2, ws0, ws1, ws2):
    d = lax.axis_index("s") * _NC + lax.axis_index("c")
    bufs = (buf0, buf1, buf2)
    rsems = (rs0, rs1, rs2)
    wsems = (ws0, ws1, ws2)

    pltpu.sync_copy(tail_hbm, tail_v)
    pltpu.sync_copy(tail_v.at[pl.ds(d * _TPAD, _NTAIL)],
                    flat_hbm.at[pl.ds(d * _VPAD + _NFULL * _CHUNK + _CHUNK2,
                                      _NTAIL)])

    sizes = [_CHUNK] * _NFULL + [_CHUNK2]
    offs = [k * _CHUNK for k in range(_NFULL)] + [_NFULL * _CHUNK]
    n = len(sizes)

    def rd(k):
        return pltpu.async_copy(
            tabT_hbm.at[d, pl.ds(offs[k], sizes[k])],
            bufs[k % 3].at[pl.ds(0, sizes[k])], rsems[k % 3])

    reads = {0: rd(0), 1: rd(1)}
    writes = {}
    for k in range(n):
        reads.pop(k).wait()
        writes[k] = pltpu.async_copy(
            bufs[k % 3].at[pl.ds(0, sizes[k])],
            flat_hbm.at[pl.ds(d * _VPAD + offs[k], sizes[k])], wsems[k % 3])
        if k + 2 < n:
            if k - 1 >= 0:
                writes.pop(k - 1).wait()
            reads[k + 2] = rd(k + 2)
    for w in writes.values():
        w.wait()


def _sc_body(title_hbm, gidx_hbm, rating_hbm, ttab_hbm, gtab_hbm, out_hbm,
             idx_v, fidx_v, trows_v, gtab_v, gidx_v, rate_v, out_v, sem):
    wid = lax.axis_index("s") * _NC + lax.axis_index("c")
    base = wid * _BW

    pltpu.sync_copy(title_hbm.at[pl.ds(base, _BW)], idx_v)

    lanes = lax.iota(jnp.int32, _L)

    dlo = lanes * _VPAD
    dhi = (lanes + _L) * _VPAD

    def fidx_body(c, carry):
        r16 = idx_v[pl.ds(c * _L, _L)]
        for j in range(_L):
            b = c * _L + j
            r = r16[j]
            fidx_v[pl.ds(b * _EMBED, _L)] = dlo + r
            fidx_v[pl.ds(b * _EMBED + _L, _L)] = dhi + r
        return carry

    lax.fori_loop(0, _BW // _L, fidx_body, 0)

    title_dma = pltpu.async_copy(ttab_hbm.at[fidx_v], trows_v, sem)

    pltpu.sync_copy(gtab_hbm, gtab_v)
    pltpu.sync_copy(gidx_hbm.at[pl.ds(base * _N_GENRES, _BW * _N_GENRES)],
                    gidx_v.at[pl.ds(0, _BW * _N_GENRES)])
    pltpu.sync_copy(rating_hbm.at[pl.ds(base, _BW)], rate_v.at[pl.ds(0, _BW)])

    def genre_body(b, carry):
        gids = gidx_v[pl.ds(b * _N_GENRES, _L)]
        g0 = jnp.zeros((_L,), jnp.float32)
        g1 = jnp.zeros((_L,), jnp.float32)
        for k in range(_N_GENRES):
            gid = gids[k]
            g0 = g0 + gtab_v[gid, pl.ds(0, _L)]
            g1 = g1 + gtab_v[gid, pl.ds(_L, _L)]
        out_v[pl.ds(b * _OUT_W + _EMBED, _L)] = g0 * (1.0 / _N_GENRES)
        out_v[pl.ds(b * _OUT_W + _EMBED + _L, _L)] = g1 * (1.0 / _N_GENRES)
        return carry

    lax.fori_loop(0, _BW, genre_body, 0)

    title_dma.wait()

    def title_body(b, carry):
        out_v[pl.ds(b * _OUT_W, _L)] = trows_v[pl.ds(b * _EMBED, _L)]
        out_v[pl.ds(b * _OUT_W + _L, _L)] = trows_v[pl.ds(b * _EMBED + _L, _L)]
        r0 = rate_v[pl.ds(b, _L)][0]
        rn = (r0 - _NORM_MEAN) * _INV_STD
        w = out_v[pl.ds(b * _OUT_W + _OUT_W - _L, _L)]
        out_v[pl.ds(b * _OUT_W + _OUT_W - _L, _L)] = jnp.where(
            lanes == _L - 1, rn, w)
        return carry

    lax.fori_loop(0, _BW, title_body, 0)

    pltpu.sync_copy(out_v, out_hbm.at[pl.ds(base * _OUT_W, _BW * _OUT_W)])


def kernel(title, book_genres, bucketized_average_rating, title_table, genre_table):
    gidx_flat = book_genres.reshape(-1)
    tabT = title_table.T
    tail72 = jnp.pad(tabT[:, _V - _NTAIL:], ((0, 0), (0, _TPAD - _NTAIL)))
    mesh = plsc.VectorSubcoreMesh(core_axis_name="c", subcore_axis_name="s")

    fmt = functools.partial(
        pl.kernel,
        mesh=mesh,
        out_type=jax.ShapeDtypeStruct((_EMBED * _VPAD,), jnp.float32),
        scratch_types=[
            pltpu.VMEM((_CHUNK,), jnp.float32),
            pltpu.VMEM((_CHUNK,), jnp.float32),
            pltpu.VMEM((_CHUNK,), jnp.float32),
            pltpu.VMEM((_EMBED * _TPAD,), jnp.float32),
            pltpu.SemaphoreType.DMA,
            pltpu.SemaphoreType.DMA,
            pltpu.SemaphoreType.DMA,
            pltpu.SemaphoreType.DMA,
            pltpu.SemaphoreType.DMA,
            pltpu.SemaphoreType.DMA,
        ],
    )(_fmt_body)
    ttab_flat = fmt(tabT, tail72.reshape(-1))

    run = functools.partial(
        pl.kernel,
        mesh=mesh,
        compiler_params=pltpu.CompilerParams(use_tc_tiling_on_sc=False),
        out_type=jax.ShapeDtypeStruct((_BATCH * _OUT_W,), jnp.float32),
        scratch_types=[
            pltpu.VMEM((_BW,), jnp.int32),
            pltpu.VMEM((_BW * _EMBED,), jnp.int32),
            pltpu.VMEM((_BW * _EMBED,), jnp.float32),
            pltpu.VMEM((_GENRE_VOCAB, _EMBED), jnp.float32),
            pltpu.VMEM((_BW * _N_GENRES + _L,), jnp.int32),
            pltpu.VMEM((_BW + _L,), jnp.float32),
            pltpu.VMEM((_BW * _OUT_W,), jnp.float32),
            pltpu.SemaphoreType.DMA,
        ],
    )(_sc_body)
    out_flat = run(title, gidx_flat, bucketized_average_rating, ttab_flat,
                   genre_table)
    return out_flat.reshape(_BATCH, _OUT_W)
